# baseline (speedup 1.0000x reference)
import functools

import jax
import jax.numpy as jnp
from jax import lax
from jax.experimental import pallas as pl
from jax.experimental.pallas import tpu as pltpu
from jax.experimental.pallas import tpu_sc as plsc

NW = 32
QB = 4096
U = 4


def _make_plane_gather(F, V, D, B, lo, npl, with_emb1):
    ppt = npl // NW
    nq = B // QB
    mesh = plsc.VectorSubcoreMesh(core_axis_name="c", subcore_axis_name="s")

    if with_emb1:
        out_type = (jax.ShapeDtypeStruct((npl, B), jnp.float32),
                    jax.ShapeDtypeStruct((F, B), jnp.float32))
    else:
        out_type = jax.ShapeDtypeStruct((npl, B), jnp.float32)

    @functools.partial(
        pl.kernel,
        mesh=mesh,
        compiler_params=pltpu.CompilerParams(needs_layout_passes=False),
        out_type=out_type,
        scratch_types=[
            pltpu.VMEM((V,), jnp.float32),
            pltpu.VMEM((B,), jnp.int32),
            pltpu.VMEM((2, QB), jnp.float32),
            pltpu.SemaphoreType.DMA,
            pltpu.SemaphoreType.DMA,
        ],
    )
    def gather(emb2_hbm, emb1_hbm, idx_hbm, *outs_and_scratch):
        if with_emb1:
            (out2_hbm, out1_hbm, plane, idx_v, obuf,
             semP, semO) = outs_and_scratch
        else:
            out2_hbm, plane, idx_v, obuf, semP, semO = outs_and_scratch
        cid = lax.axis_index("c")
        sid = lax.axis_index("s")
        w = sid * 2 + cid
        is_e1 = w >= NW - F
        f_e1 = w - (NW - F)

        def start_P(tab, row):
            pltpu.async_copy(tab.at[row], plane, semP)

        def wait_P():
            pltpu.make_async_copy(emb2_hbm.at[0], plane, semP).wait()

        def wait_w():
            pltpu.make_async_copy(obuf.at[0],
                                  out2_hbm.at[0].at[pl.ds(0, QB)],
                                  semO).wait()

        def do_quarter(q, out_tab, orow):
            par = q % 2

            @plsc.parallel_loop(0, QB // 16, unroll=U)
            def _(i):
                o = pl.multiple_of(i * 16, 16)
                iv = idx_v[pl.ds(q * QB + o, 16)]
                obuf[par, pl.ds(o, 16)] = plsc.load_gather(plane, [iv])

            pltpu.async_copy(obuf.at[par],
                             out_tab.at[orow].at[pl.ds(q * QB, QB)], semO)

        def plane_proc(out_tab, orow, warm):
            for q in range(nq):
                if q < 2:
                    @pl.when(warm)
                    def _():
                        wait_w()
                else:
                    wait_w()
                do_quarter(q, out_tab, orow)

        def body(j, fprev):
            p = lo + w * ppt + j
            f = p // D
            nxt = j + 1

            @pl.when(f != fprev)
            def _():
                pltpu.sync_copy(idx_hbm.at[f], idx_v)

            wait_P()
            plane_proc(out2_hbm, w * ppt + j, j > 0)

            @pl.when(nxt < ppt)
            def _():
                start_P(emb2_hbm, p + 1)
            if with_emb1:
                @pl.when(jnp.logical_and(nxt == ppt, is_e1))
                def _():
                    start_P(emb1_hbm, f_e1)
            return f

        start_P(emb2_hbm, lo + w * ppt)
        lax.fori_loop(0, ppt, body, -1)

        if with_emb1:
            @pl.when(is_e1)
            def _():
                pltpu.sync_copy(idx_hbm.at[f_e1], idx_v)
                wait_P()
                plane_proc(out1_hbm, f_e1, w >= 0)

        wait_w()
        wait_w()

    return gather


def _tc1_body(catA_ref, W0eA_ref, S16A_ref, onesA_ref,
              h0p_ref, semb_ref, ssq_ref):
    prec = lax.Precision.DEFAULT
    dn = (((0,), (0,)), ((), ()))
    catA = catA_ref[...]
    h0p_ref[...] = lax.dot_general(catA, W0eA_ref[...], dn, precision=prec)
    semb_ref[...] = lax.dot_general(catA, S16A_ref[...], dn, precision=prec)
    ssq_ref[...] = lax.dot_general(onesA_ref[...], catA * catA, dn,
                                   precision=prec)


def _tc2_body(cont_ref, catB_ref, g1T_ref, h0p_ref, semb_ref, ssq_ref,
              Wc_ref, W0c_ref, W0eB_ref, b0_ref, W1_ref, b1_ref, Wh_ref,
              S16B_ref, onesB_ref, onesF_ref, onesD_ref, sc_ref, out_ref):
    prec = lax.Precision.DEFAULT
    dn = (((0,), (0,)), ((), ()))
    dnT = (((0,), (1,)), ((), ()))
    cont = cont_ref[...]
    catB = catB_ref[...]
    g1T = g1T_ref[...]
    b_cont = sc_ref[0]
    b_out = sc_ref[1]
    w_fm = sc_ref[2]
    fm1 = (lax.dot_general(Wc_ref[...], cont, dnT, precision=prec)
           + lax.dot_general(onesF_ref[...], g1T, dn, precision=prec)
           + b_cont)
    sum_emb = semb_ref[...] + lax.dot_general(catB, S16B_ref[...], dn,
                                              precision=prec)
    square_sum = lax.dot_general(onesD_ref[...], sum_emb * sum_emb, dnT,
                                 precision=prec)
    sumsq = ssq_ref[...] + lax.dot_general(onesB_ref[...], catB * catB, dn,
                                           precision=prec)
    fm = fm1 + 0.5 * (square_sum - sumsq)
    h = jnp.maximum(h0p_ref[...]
                    + jnp.dot(cont, W0c_ref[...], precision=prec)
                    + lax.dot_general(catB, W0eB_ref[...], dn, precision=prec)
                    + b0_ref[...], 0.0)
    h = jnp.maximum(jnp.dot(h, W1_ref[...], precision=prec) + b1_ref[...], 0.0)
    out_ref[...] = (lax.dot_general(Wh_ref[...], h, dnT, precision=prec)
                    + fm * w_fm + b_out)


def kernel(continuous, categorical, emb1, emb2, W_cont, b_cont, W0, b0, W1,
           b1, W_out, b_out):
    F, V, D = emb2.shape
    B, C = continuous.shape
    H0 = W0.shape[1]
    H1 = W1.shape[1]
    FD = F * D
    npA = 352
    npB = FD - npA

    emb2_pl = emb2.transpose(0, 2, 1).reshape(FD, V)
    emb1_pl = emb1.transpose(0, 2, 1).reshape(F, V)
    idx = categorical.reshape(F, B).astype(jnp.int32)

    gatherA = _make_plane_gather(F, V, D, B, 0, npA, False)
    gatherB = _make_plane_gather(F, V, D, B, npA, npB, True)
    catA = gatherA(emb2_pl, emb1_pl, idx)
    catB, g1T = gatherB(emb2_pl, emb1_pl, idx)

    S16 = jnp.tile(jnp.eye(D, dtype=jnp.float32), (F, 1))
    sc = jnp.concatenate([b_cont, b_out, W_out[0]]).astype(jnp.float32)
    W0c = W0[:C]
    W0eA = W0[C:C + npA]
    W0eB = W0[C + npA:]
    S16A = S16[:npA]
    S16B = S16[npA:]
    onesA = jnp.ones((npA, 1), jnp.float32)
    onesB = jnp.ones((npB, 1), jnp.float32)
    onesF = jnp.ones((F, 1), jnp.float32)
    onesD = jnp.ones((D, 1), jnp.float32)
    Wh = W_out[1:]

    BB = 2048
    rep = lambda i: (0, 0)
    col = lambda i: (0, i)
    row = lambda i: (i, 0)
    h0p, semb, ssq = pl.pallas_call(
        _tc1_body,
        grid=(B // BB,),
        in_specs=[
            pl.BlockSpec((npA, BB), col),
            pl.BlockSpec((npA, H0), rep),
            pl.BlockSpec((npA, D), rep),
            pl.BlockSpec((npA, 1), rep),
        ],
        out_specs=[
            pl.BlockSpec((BB, H0), row),
            pl.BlockSpec((BB, D), row),
            pl.BlockSpec((1, BB), col),
        ],
        out_shape=[
            jax.ShapeDtypeStruct((B, H0), jnp.float32),
            jax.ShapeDtypeStruct((B, D), jnp.float32),
            jax.ShapeDtypeStruct((1, B), jnp.float32),
        ],
    )(catA, W0eA, S16A, onesA)

    out = pl.pallas_call(
        _tc2_body,
        grid=(B // BB,),
        in_specs=[
            pl.BlockSpec((BB, C), row),
            pl.BlockSpec((npB, BB), col),
            pl.BlockSpec((F, BB), col),
            pl.BlockSpec((BB, H0), row),
            pl.BlockSpec((BB, D), row),
            pl.BlockSpec((1, BB), col),
            pl.BlockSpec((C, 1), rep),
            pl.BlockSpec((C, H0), rep),
            pl.BlockSpec((npB, H0), rep),
            pl.BlockSpec((1, H0), rep),
            pl.BlockSpec((H0, H1), rep),
            pl.BlockSpec((1, H1), rep),
            pl.BlockSpec((H1, 1), rep),
            pl.BlockSpec((npB, D), rep),
            pl.BlockSpec((npB, 1), rep),
            pl.BlockSpec((F, 1), rep),
            pl.BlockSpec((D, 1), rep),
            pl.BlockSpec(memory_space=pltpu.SMEM),
        ],
        out_specs=pl.BlockSpec((1, BB), col),
        out_shape=jax.ShapeDtypeStruct((1, B), jnp.float32),
    )(continuous, catB, g1T, h0p, semb, ssq, W_cont, W0c, W0eB,
      b0.reshape(1, H0), W1, b1.reshape(1, H1), Wh, S16B, onesB, onesF,
      onesD, sc)
    return out.reshape(B, 1)


# speedup vs baseline: 1.0045x; 1.0045x over previous
import functools

import jax
import jax.numpy as jnp
from jax import lax
from jax.experimental import pallas as pl
from jax.experimental.pallas import tpu as pltpu
from jax.experimental.pallas import tpu_sc as plsc

NW = 32
QB = 4096
U = 4


def _make_plane_gather(F, V, D, B, lo, npl, with_emb1):
    ppt = npl // NW
    nq = B // QB
    mesh = plsc.VectorSubcoreMesh(core_axis_name="c", subcore_axis_name="s")

    if with_emb1:
        out_type = (jax.ShapeDtypeStruct((npl, B), jnp.float32),
                    jax.ShapeDtypeStruct((F, B), jnp.float32))
    else:
        out_type = jax.ShapeDtypeStruct((npl, B), jnp.float32)

    @functools.partial(
        pl.kernel,
        mesh=mesh,
        compiler_params=pltpu.CompilerParams(needs_layout_passes=False),
        out_type=out_type,
        scratch_types=[
            pltpu.VMEM((V,), jnp.float32),
            pltpu.VMEM((B,), jnp.int32),
            pltpu.VMEM((2, QB), jnp.float32),
            pltpu.SemaphoreType.DMA,
            pltpu.SemaphoreType.DMA,
            pltpu.SemaphoreType.DMA,
        ],
    )
    def gather(emb2_hbm, emb1_hbm, idx_hbm, *outs_and_scratch):
        if with_emb1:
            (out2_hbm, out1_hbm, plane, idx_v, obuf,
             semP, semO0, semO1) = outs_and_scratch
        else:
            (out2_hbm, plane, idx_v, obuf,
             semP, semO0, semO1) = outs_and_scratch
        semO = (semO0, semO1)
        cid = lax.axis_index("c")
        sid = lax.axis_index("s")
        w = sid * 2 + cid
        is_e1 = w >= NW - F
        f_e1 = w - (NW - F)

        def start_P(tab, row):
            pltpu.async_copy(tab.at[row], plane, semP)

        def wait_P():
            pltpu.make_async_copy(emb2_hbm.at[0], plane, semP).wait()

        def wait_w(par):
            pltpu.make_async_copy(obuf.at[0],
                                  out2_hbm.at[0].at[pl.ds(0, QB)],
                                  semO[par]).wait()

        def do_quarter(q, out_tab, orow):
            par = q % 2

            @plsc.parallel_loop(0, QB // 16, unroll=U)
            def _(i):
                o = pl.multiple_of(i * 16, 16)
                iv = idx_v[pl.ds(q * QB + o, 16)]
                obuf[par, pl.ds(o, 16)] = plsc.load_gather(plane, [iv])

            pltpu.async_copy(obuf.at[par],
                             out_tab.at[orow].at[pl.ds(q * QB, QB)],
                             semO[par])

        def plane_proc(out_tab, orow, warm):
            for q in range(nq):
                if q < 2:
                    @pl.when(warm)
                    def _():
                        wait_w(q % 2)
                else:
                    wait_w(q % 2)
                do_quarter(q, out_tab, orow)

        def body(j, fprev):
            p = lo + w * ppt + j
            f = p // D
            nxt = j + 1

            @pl.when(f != fprev)
            def _():
                pltpu.sync_copy(idx_hbm.at[f], idx_v)

            wait_P()
            plane_proc(out2_hbm, w * ppt + j, j > 0)

            @pl.when(nxt < ppt)
            def _():
                start_P(emb2_hbm, p + 1)
            if with_emb1:
                @pl.when(jnp.logical_and(nxt == ppt, is_e1))
                def _():
                    start_P(emb1_hbm, f_e1)
            return f

        start_P(emb2_hbm, lo + w * ppt)
        lax.fori_loop(0, ppt, body, -1)

        if with_emb1:
            @pl.when(is_e1)
            def _():
                pltpu.sync_copy(idx_hbm.at[f_e1], idx_v)
                wait_P()
                plane_proc(out1_hbm, f_e1, w >= 0)

        wait_w(0)
        wait_w(1)

    return gather


def _tc1_body(catA_ref, W0eA_ref, S16A_ref, onesA_ref,
              h0p_ref, semb_ref, ssq_ref):
    prec = lax.Precision.DEFAULT
    dn = (((0,), (0,)), ((), ()))
    catA = catA_ref[...]
    h0p_ref[...] = lax.dot_general(catA, W0eA_ref[...], dn, precision=prec)
    semb_ref[...] = lax.dot_general(catA, S16A_ref[...], dn, precision=prec)
    ssq_ref[...] = lax.dot_general(onesA_ref[...], catA * catA, dn,
                                   precision=prec)


def _tc2_body(cont_ref, catB_ref, g1T_ref, h0p_ref, semb_ref, ssq_ref,
              Wc_ref, W0c_ref, W0eB_ref, b0_ref, W1_ref, b1_ref, Wh_ref,
              S16B_ref, onesB_ref, onesF_ref, onesD_ref, sc_ref, out_ref):
    prec = lax.Precision.DEFAULT
    dn = (((0,), (0,)), ((), ()))
    dnT = (((0,), (1,)), ((), ()))
    cont = cont_ref[...]
    catB = catB_ref[...]
    g1T = g1T_ref[...]
    b_cont = sc_ref[0]
    b_out = sc_ref[1]
    w_fm = sc_ref[2]
    fm1 = (lax.dot_general(Wc_ref[...], cont, dnT, precision=prec)
           + lax.dot_general(onesF_ref[...], g1T, dn, precision=prec)
           + b_cont)
    sum_emb = semb_ref[...] + lax.dot_general(catB, S16B_ref[...], dn,
                                              precision=prec)
    square_sum = lax.dot_general(onesD_ref[...], sum_emb * sum_emb, dnT,
                                 precision=prec)
    sumsq = ssq_ref[...] + lax.dot_general(onesB_ref[...], catB * catB, dn,
                                           precision=prec)
    fm = fm1 + 0.5 * (square_sum - sumsq)
    h = jnp.maximum(h0p_ref[...]
                    + jnp.dot(cont, W0c_ref[...], precision=prec)
                    + lax.dot_general(catB, W0eB_ref[...], dn, precision=prec)
                    + b0_ref[...], 0.0)
    h = jnp.maximum(jnp.dot(h, W1_ref[...], precision=prec) + b1_ref[...], 0.0)
    out_ref[...] = (lax.dot_general(Wh_ref[...], h, dnT, precision=prec)
                    + fm * w_fm + b_out)


def kernel(continuous, categorical, emb1, emb2, W_cont, b_cont, W0, b0, W1,
           b1, W_out, b_out):
    F, V, D = emb2.shape
    B, C = continuous.shape
    H0 = W0.shape[1]
    H1 = W1.shape[1]
    FD = F * D
    npA = 320
    npB = FD - npA

    emb2_pl = emb2.transpose(0, 2, 1).reshape(FD, V)
    emb1_pl = emb1.transpose(0, 2, 1).reshape(F, V)
    idx = categorical.reshape(F, B).astype(jnp.int32)

    gatherA = _make_plane_gather(F, V, D, B, 0, npA, False)
    gatherB = _make_plane_gather(F, V, D, B, npA, npB, True)
    catA = gatherA(emb2_pl, emb1_pl, idx)
    catB, g1T = gatherB(emb2_pl, emb1_pl, idx)

    S16 = jnp.tile(jnp.eye(D, dtype=jnp.float32), (F, 1))
    sc = jnp.concatenate([b_cont, b_out, W_out[0]]).astype(jnp.float32)
    W0c = W0[:C]
    W0eA = W0[C:C + npA]
    W0eB = W0[C + npA:]
    S16A = S16[:npA]
    S16B = S16[npA:]
    onesA = jnp.ones((npA, 1), jnp.float32)
    onesB = jnp.ones((npB, 1), jnp.float32)
    onesF = jnp.ones((F, 1), jnp.float32)
    onesD = jnp.ones((D, 1), jnp.float32)
    Wh = W_out[1:]

    BB = 2048
    rep = lambda i: (0, 0)
    col = lambda i: (0, i)
    row = lambda i: (i, 0)
    h0p, semb, ssq = pl.pallas_call(
        _tc1_body,
        grid=(B // BB,),
        in_specs=[
            pl.BlockSpec((npA, BB), col),
            pl.BlockSpec((npA, H0), rep),
            pl.BlockSpec((npA, D), rep),
            pl.BlockSpec((npA, 1), rep),
        ],
        out_specs=[
            pl.BlockSpec((BB, H0), row),
            pl.BlockSpec((BB, D), row),
            pl.BlockSpec((1, BB), col),
        ],
        out_shape=[
            jax.ShapeDtypeStruct((B, H0), jnp.float32),
            jax.ShapeDtypeStruct((B, D), jnp.float32),
            jax.ShapeDtypeStruct((1, B), jnp.float32),
        ],
    )(catA, W0eA, S16A, onesA)

    out = pl.pallas_call(
        _tc2_body,
        grid=(B // BB,),
        in_specs=[
            pl.BlockSpec((BB, C), row),
            pl.BlockSpec((npB, BB), col),
            pl.BlockSpec((F, BB), col),
            pl.BlockSpec((BB, H0), row),
            pl.BlockSpec((BB, D), row),
            pl.BlockSpec((1, BB), col),
            pl.BlockSpec((C, 1), rep),
            pl.BlockSpec((C, H0), rep),
            pl.BlockSpec((npB, H0), rep),
            pl.BlockSpec((1, H0), rep),
            pl.BlockSpec((H0, H1), rep),
            pl.BlockSpec((1, H1), rep),
            pl.BlockSpec((H1, 1), rep),
            pl.BlockSpec((npB, D), rep),
            pl.BlockSpec((npB, 1), rep),
            pl.BlockSpec((F, 1), rep),
            pl.BlockSpec((D, 1), rep),
            pl.BlockSpec(memory_space=pltpu.SMEM),
        ],
        out_specs=pl.BlockSpec((1, BB), col),
        out_shape=jax.ShapeDtypeStruct((1, B), jnp.float32),
    )(continuous, catB, g1T, h0p, semb, ssq, W_cont, W0c, W0eB,
      b0.reshape(1, H0), W1, b1.reshape(1, H1), Wh, S16B, onesB, onesF,
      onesD, sc)
    return out.reshape(B, 1)


# speedup vs baseline: 1.0192x; 1.0147x over previous
import functools

import jax
import jax.numpy as jnp
from jax import lax
from jax.experimental import pallas as pl
from jax.experimental.pallas import tpu as pltpu
from jax.experimental.pallas import tpu_sc as plsc

NW = 32
QB = 4096
U = 8


def _make_plane_gather(F, V, D, B, lo, npl, with_emb1):
    ppt = npl // NW
    nq = B // QB
    mesh = plsc.VectorSubcoreMesh(core_axis_name="c", subcore_axis_name="s")

    if with_emb1:
        out_type = (jax.ShapeDtypeStruct((npl, B), jnp.float32),
                    jax.ShapeDtypeStruct((F, B), jnp.float32))
    else:
        out_type = jax.ShapeDtypeStruct((npl, B), jnp.float32)

    @functools.partial(
        pl.kernel,
        mesh=mesh,
        compiler_params=pltpu.CompilerParams(needs_layout_passes=False),
        out_type=out_type,
        scratch_types=[
            pltpu.VMEM((V,), jnp.float32),
            pltpu.VMEM((B,), jnp.int32),
            pltpu.VMEM((2, QB), jnp.float32),
            pltpu.SemaphoreType.DMA,
            pltpu.SemaphoreType.DMA,
            pltpu.SemaphoreType.DMA,
        ],
    )
    def gather(emb2_hbm, emb1_hbm, idx_hbm, *outs_and_scratch):
        if with_emb1:
            (out2_hbm, out1_hbm, plane, idx_v, obuf,
             semP, semO0, semO1) = outs_and_scratch
        else:
            (out2_hbm, plane, idx_v, obuf,
             semP, semO0, semO1) = outs_and_scratch
        semO = (semO0, semO1)
        cid = lax.axis_index("c")
        sid = lax.axis_index("s")
        w = sid * 2 + cid
        is_e1 = w >= NW - F
        f_e1 = w - (NW - F)

        def start_P(tab, row):
            pltpu.async_copy(tab.at[row], plane, semP)

        def wait_P():
            pltpu.make_async_copy(emb2_hbm.at[0], plane, semP).wait()

        def wait_w(par):
            pltpu.make_async_copy(obuf.at[0],
                                  out2_hbm.at[0].at[pl.ds(0, QB)],
                                  semO[par]).wait()

        def do_quarter(q, out_tab, orow):
            par = q % 2

            @plsc.parallel_loop(0, QB // 16, unroll=U)
            def _(i):
                o = pl.multiple_of(i * 16, 16)
                iv = idx_v[pl.ds(q * QB + o, 16)]
                obuf[par, pl.ds(o, 16)] = plsc.load_gather(plane, [iv])

            pltpu.async_copy(obuf.at[par],
                             out_tab.at[orow].at[pl.ds(q * QB, QB)],
                             semO[par])

        def plane_proc(out_tab, orow, warm):
            for q in range(nq):
                if q < 2:
                    @pl.when(warm)
                    def _():
                        wait_w(q % 2)
                else:
                    wait_w(q % 2)
                do_quarter(q, out_tab, orow)

        def body(j, fprev):
            p = lo + w * ppt + j
            f = p // D
            nxt = j + 1

            @pl.when(f != fprev)
            def _():
                pltpu.sync_copy(idx_hbm.at[f], idx_v)

            wait_P()
            plane_proc(out2_hbm, w * ppt + j, j > 0)

            @pl.when(nxt < ppt)
            def _():
                start_P(emb2_hbm, p + 1)
            if with_emb1:
                @pl.when(jnp.logical_and(nxt == ppt, is_e1))
                def _():
                    start_P(emb1_hbm, f_e1)
            return f

        start_P(emb2_hbm, lo + w * ppt)
        lax.fori_loop(0, ppt, body, -1)

        if with_emb1:
            @pl.when(is_e1)
            def _():
                pltpu.sync_copy(idx_hbm.at[f_e1], idx_v)
                wait_P()
                plane_proc(out1_hbm, f_e1, w >= 0)

        wait_w(0)
        wait_w(1)

    return gather


def _tc1_body(catA_ref, W0eA_ref, S16A_ref, onesA_ref,
              h0p_ref, semb_ref, ssq_ref):
    prec = lax.Precision.DEFAULT
    dn = (((0,), (0,)), ((), ()))
    catA = catA_ref[...]
    h0p_ref[...] = lax.dot_general(catA, W0eA_ref[...], dn, precision=prec)
    semb_ref[...] = lax.dot_general(catA, S16A_ref[...], dn, precision=prec)
    ssq_ref[...] = lax.dot_general(onesA_ref[...], catA * catA, dn,
                                   precision=prec)


def _tc2_body(cont_ref, catB_ref, g1T_ref, h0p_ref, semb_ref, ssq_ref,
              Wc_ref, W0c_ref, W0eB_ref, b0_ref, W1_ref, b1_ref, Wh_ref,
              S16B_ref, onesB_ref, onesF_ref, onesD_ref, sc_ref, out_ref):
    prec = lax.Precision.DEFAULT
    dn = (((0,), (0,)), ((), ()))
    dnT = (((0,), (1,)), ((), ()))
    cont = cont_ref[...]
    catB = catB_ref[...]
    g1T = g1T_ref[...]
    b_cont = sc_ref[0]
    b_out = sc_ref[1]
    w_fm = sc_ref[2]
    fm1 = (lax.dot_general(Wc_ref[...], cont, dnT, precision=prec)
           + lax.dot_general(onesF_ref[...], g1T, dn, precision=prec)
           + b_cont)
    sum_emb = semb_ref[...] + lax.dot_general(catB, S16B_ref[...], dn,
                                              precision=prec)
    square_sum = lax.dot_general(onesD_ref[...], sum_emb * sum_emb, dnT,
                                 precision=prec)
    sumsq = ssq_ref[...] + lax.dot_general(onesB_ref[...], catB * catB, dn,
                                           precision=prec)
    fm = fm1 + 0.5 * (square_sum - sumsq)
    h = jnp.maximum(h0p_ref[...]
                    + jnp.dot(cont, W0c_ref[...], precision=prec)
                    + lax.dot_general(catB, W0eB_ref[...], dn, precision=prec)
                    + b0_ref[...], 0.0)
    h = jnp.maximum(jnp.dot(h, W1_ref[...], precision=prec) + b1_ref[...], 0.0)
    out_ref[...] = (lax.dot_general(Wh_ref[...], h, dnT, precision=prec)
                    + fm * w_fm + b_out)


def kernel(continuous, categorical, emb1, emb2, W_cont, b_cont, W0, b0, W1,
           b1, W_out, b_out):
    F, V, D = emb2.shape
    B, C = continuous.shape
    H0 = W0.shape[1]
    H1 = W1.shape[1]
    FD = F * D
    npA = 320
    npB = FD - npA

    emb2_pl = emb2.transpose(0, 2, 1).reshape(FD, V)
    emb1_pl = emb1.transpose(0, 2, 1).reshape(F, V)
    idx = categorical.reshape(F, B).astype(jnp.int32)

    gatherA = _make_plane_gather(F, V, D, B, 0, npA, False)
    gatherB = _make_plane_gather(F, V, D, B, npA, npB, True)
    catA = gatherA(emb2_pl, emb1_pl, idx)
    catB, g1T = gatherB(emb2_pl, emb1_pl, idx)

    S16 = jnp.tile(jnp.eye(D, dtype=jnp.float32), (F, 1))
    sc = jnp.concatenate([b_cont, b_out, W_out[0]]).astype(jnp.float32)
    W0c = W0[:C]
    W0eA = W0[C:C + npA]
    W0eB = W0[C + npA:]
    S16A = S16[:npA]
    S16B = S16[npA:]
    onesA = jnp.ones((npA, 1), jnp.float32)
    onesB = jnp.ones((npB, 1), jnp.float32)
    onesF = jnp.ones((F, 1), jnp.float32)
    onesD = jnp.ones((D, 1), jnp.float32)
    Wh = W_out[1:]

    BB = 4096
    rep = lambda i: (0, 0)
    col = lambda i: (0, i)
    row = lambda i: (i, 0)
    h0p, semb, ssq = pl.pallas_call(
        _tc1_body,
        grid=(B // BB,),
        in_specs=[
            pl.BlockSpec((npA, BB), col),
            pl.BlockSpec((npA, H0), rep),
            pl.BlockSpec((npA, D), rep),
            pl.BlockSpec((npA, 1), rep),
        ],
        out_specs=[
            pl.BlockSpec((BB, H0), row),
            pl.BlockSpec((BB, D), row),
            pl.BlockSpec((1, BB), col),
        ],
        out_shape=[
            jax.ShapeDtypeStruct((B, H0), jnp.float32),
            jax.ShapeDtypeStruct((B, D), jnp.float32),
            jax.ShapeDtypeStruct((1, B), jnp.float32),
        ],
    )(catA, W0eA, S16A, onesA)

    out = pl.pallas_call(
        _tc2_body,
        grid=(B // BB,),
        in_specs=[
            pl.BlockSpec((BB, C), row),
            pl.BlockSpec((npB, BB), col),
            pl.BlockSpec((F, BB), col),
            pl.BlockSpec((BB, H0), row),
            pl.BlockSpec((BB, D), row),
            pl.BlockSpec((1, BB), col),
            pl.BlockSpec((C, 1), rep),
            pl.BlockSpec((C, H0), rep),
            pl.BlockSpec((npB, H0), rep),
            pl.BlockSpec((1, H0), rep),
            pl.BlockSpec((H0, H1), rep),
            pl.BlockSpec((1, H1), rep),
            pl.BlockSpec((H1, 1), rep),
            pl.BlockSpec((npB, D), rep),
            pl.BlockSpec((npB, 1), rep),
            pl.BlockSpec((F, 1), rep),
            pl.BlockSpec((D, 1), rep),
            pl.BlockSpec(memory_space=pltpu.SMEM),
        ],
        out_specs=pl.BlockSpec((1, BB), col),
        out_shape=jax.ShapeDtypeStruct((1, B), jnp.float32),
    )(continuous, catB, g1T, h0p, semb, ssq, W_cont, W0c, W0eB,
      b0.reshape(1, H0), W1, b1.reshape(1, H1), Wh, S16B, onesB, onesF,
      onesD, sc)
    return out.reshape(B, 1)
